# baseline (speedup 1.0000x reference)
import jax
import jax.numpy as jnp
from jax import lax
from jax.experimental import pallas as pl

_T = 320
_C = 1024
_K = 48
_KPAD = 128
_BO = 256
_G = _C // _BO


def _conv_body(w_ref, x_ref, b_ref, o_ref):
    acc = jnp.dot(w_ref[...], x_ref[...], preferred_element_type=jnp.float32)
    o_ref[...] = jnp.maximum(acc + b_ref[...], 0.0)


def _conv_relu(xcat, wr, b):
    return pl.pallas_call(
        _conv_body,
        grid=(_G,),
        in_specs=[
            pl.BlockSpec((_BO, 3 * _C), lambda i: (i, 0)),
            pl.BlockSpec((3 * _C, _T), lambda i: (0, 0)),
            pl.BlockSpec((_BO, 1), lambda i: (i, 0)),
        ],
        out_specs=pl.BlockSpec((_BO, _T), lambda i: (i, 0)),
        out_shape=jax.ShapeDtypeStruct((_C, _T), jnp.float32),
    )(wr, xcat, b)


def _tail_body(h_ref, ws_ref, bs_ref, o_ref):
    h = h_ref[...]
    ws = ws_ref[...]
    s = jnp.dot(ws.astype(jnp.bfloat16), h.astype(jnp.bfloat16),
                preferred_element_type=jnp.float32) + bs_ref[...]
    csum = jnp.sum(h, axis=0, keepdims=True)

    u_iota = lax.broadcasted_iota(jnp.int32, (_T, _T), 0)
    t_iota = lax.broadcasted_iota(jnp.int32, (_T, _T), 1)
    eye = (u_iota == t_iota).astype(jnp.float32)
    diag = eye * s
    scol = jnp.dot(diag, jnp.ones((_T, _T), jnp.float32),
                   preferred_element_type=jnp.float32,
                   precision=lax.Precision.HIGHEST)
    cmp = (scol >= s).astype(jnp.float32)
    cnt = jnp.sum(cmp, axis=0, keepdims=True)
    low = jnp.max(jnp.where(cnt >= _K, s, -1e30), axis=1, keepdims=True)

    maskf = (s >= low).astype(jnp.float32)
    nz = maskf * (csum != 0.0).astype(jnp.float32)
    ltm = (u_iota <= t_iota).astype(jnp.float32)
    prefix = jnp.dot(nz, ltm, preferred_element_type=jnp.float32,
                     precision=lax.Precision.HIGHEST)
    count = jnp.sum(nz, axis=1, keepdims=True)

    j_iota = lax.broadcasted_iota(jnp.int32, (_KPAD, _T), 0).astype(jnp.float32)
    t2 = lax.broadcasted_iota(jnp.int32, (_KPAD, _T), 1)
    sel = ((prefix == j_iota + 1.0) & (nz > 0.0)).astype(jnp.float32)
    pad = ((j_iota + 1.0 > count) & (t2 == 0)).astype(jnp.float32) * maskf
    pt = sel + pad
    o_ref[...] = lax.dot_general(h, pt, (((1,), (1,)), ((), ())),
                                 preferred_element_type=jnp.float32,
                                 precision=lax.Precision.HIGHEST)


def _tail(h2, ws_row, bs11):
    return pl.pallas_call(
        _tail_body,
        in_specs=[
            pl.BlockSpec((_C, _T), lambda: (0, 0)),
            pl.BlockSpec((1, _C), lambda: (0, 0)),
            pl.BlockSpec((1, 1), lambda: (0, 0)),
        ],
        out_specs=pl.BlockSpec((_C, _KPAD), lambda: (0, 0)),
        out_shape=jax.ShapeDtypeStruct((_C, _KPAD), jnp.float32),
    )(h2, ws_row, bs11)


def _shift_cat(x):
    xpad = jnp.pad(x, ((0, 0), (1, 1)))
    return jnp.concatenate(
        [xpad[:, 0:_T], xpad[:, 1:_T + 1], xpad[:, 2:_T + 2]], axis=0)


def kernel(x, W1, b1, W2, b2, Ws, bs):
    X = x[0]
    w1r = jnp.transpose(W1, (0, 2, 1)).reshape(_C, 3 * _C)
    w2r = jnp.transpose(W2, (0, 2, 1)).reshape(_C, 3 * _C)
    h1 = _conv_relu(_shift_cat(X), w1r, b1.reshape(_C, 1))
    h2 = _conv_relu(_shift_cat(h1), w2r, b2.reshape(_C, 1))
    out = _tail(h2, Ws.reshape(1, _C), bs.reshape(1, 1))
    return out[None, :, :_K]
